# Initial kernel scaffold; baseline (speedup 1.0000x reference)
import jax
import jax.numpy as jnp
from jax.experimental import pallas as pl


def kernel(x, edge_index, W1, b1, W2, b2, Wd, bd):
    raise NotImplementedError("write your pallas kernel here")


# speedup vs baseline: 12.6591x; 12.6591x over previous
import functools

import jax
import jax.numpy as jnp
from jax import lax
from jax.experimental import pallas as pl
from jax.experimental.pallas import tpu as pltpu
from jax.experimental.pallas import tpu_sc as plsc

F32 = jnp.float32
I32 = jnp.int32

_N_CORES = 2
_N_SUB = 16
_NW = _N_CORES * _N_SUB
_K = 128
_TC_BLK = 512
_DEG_W = 16


def _ceil_to(v, m):
    return (v + m - 1) // m * m


@functools.lru_cache(maxsize=None)
def _make_deg(n_pad, e_pad):
    ept = e_pad // _NW
    nchunk = ept // _K
    rows_pt = n_pad // _N_SUB
    rseg = rows_pt // _K
    mesh = plsc.VectorSubcoreMesh(core_axis_name="c", subcore_axis_name="s")

    def body(dst_hbm, out_hbm, idx_d, ones_v, zbuf, acc):
        c = lax.axis_index("c")
        s = lax.axis_index("s")
        wid = c * _N_SUB + s

        def fill(i, _):
            ones_v[i, :] = jnp.ones((16,), F32)
            zbuf[i, :] = jnp.zeros((16,), F32)
            return 0

        lax.fori_loop(0, _K, fill, 0)

        def zinit(i, _):
            pltpu.sync_copy(zbuf, acc.at[pl.ds(s * rows_pt + i * _K, _K)])
            return 0

        lax.fori_loop(0, rseg, zinit, 0)
        plsc.subcore_barrier()

        base = wid * ept

        def chunk(j, _):
            pltpu.sync_copy(dst_hbm.at[pl.ds(base + j * _K, _K)], idx_d)
            pltpu.sync_copy(ones_v, acc.at[idx_d], add=True)
            return 0

        lax.fori_loop(0, nchunk, chunk, 0)
        plsc.subcore_barrier()

        def wb(i, _):
            r0 = s * rows_pt + i * _K
            pltpu.sync_copy(acc.at[pl.ds(r0, _K)], zbuf)
            pltpu.sync_copy(zbuf, out_hbm.at[c, pl.ds(r0, _K)])
            return 0

        lax.fori_loop(0, rseg, wb, 0)

    return pl.kernel(
        body,
        out_type=jax.ShapeDtypeStruct((_N_CORES, n_pad, _DEG_W), F32),
        mesh=mesh,
        compiler_params=pltpu.CompilerParams(use_tc_tiling_on_sc=False),
        scratch_types=[
            pltpu.VMEM((_K,), I32),
            pltpu.VMEM((_K, _DEG_W), F32),
            pltpu.VMEM((_K, _DEG_W), F32),
            pltpu.VMEM_SHARED((n_pad, _DEG_W), F32),
        ],
    )


@functools.lru_cache(maxsize=None)
def _make_agg(n_pad, e_pad, c_dim):
    ept = e_pad // _NW
    nchunk = ept // _K
    rows_pt = n_pad // _N_SUB
    rseg = rows_pt // _K
    zero_row = n_pad - _K
    mesh = plsc.VectorSubcoreMesh(core_axis_name="c", subcore_axis_name="s")

    def body(y_hbm, src_hbm, dst_hbm, out_hbm, idx_s, idx_d, rows, acc, sem):
        c = lax.axis_index("c")
        s = lax.axis_index("s")
        wid = c * _N_SUB + s

        def seed(i, _):
            dst0 = s * rows_pt + i * _K
            src0 = jnp.where(c == 0, dst0, zero_row)
            pltpu.sync_copy(y_hbm.at[pl.ds(src0, _K)], rows)
            pltpu.sync_copy(rows, acc.at[pl.ds(dst0, _K)])
            return 0

        lax.fori_loop(0, rseg, seed, 0)
        plsc.subcore_barrier()

        base = wid * ept

        def chunk(j, _):
            eb = base + j * _K
            pltpu.sync_copy(src_hbm.at[pl.ds(eb, _K)], idx_s)
            pltpu.sync_copy(dst_hbm.at[pl.ds(eb, _K)], idx_d)
            pltpu.async_copy(y_hbm.at[idx_s], rows, sem).wait()
            pltpu.sync_copy(rows, acc.at[idx_d], add=True)
            return 0

        lax.fori_loop(0, nchunk, chunk, 0)
        plsc.subcore_barrier()

        def wb(i, _):
            r0 = s * rows_pt + i * _K
            pltpu.sync_copy(acc.at[pl.ds(r0, _K)], rows)
            pltpu.sync_copy(rows, out_hbm.at[c, pl.ds(r0, _K)])
            return 0

        lax.fori_loop(0, rseg, wb, 0)

    return pl.kernel(
        body,
        out_type=jax.ShapeDtypeStruct((_N_CORES, n_pad, c_dim), F32),
        mesh=mesh,
        compiler_params=pltpu.CompilerParams(use_tc_tiling_on_sc=False),
        scratch_types=[
            pltpu.VMEM((_K,), I32),
            pltpu.VMEM((_K,), I32),
            pltpu.VMEM((_K, c_dim), F32),
            pltpu.VMEM_SHARED((n_pad, c_dim), F32),
            pltpu.SemaphoreType.DMA,
        ],
    )


def _dinv_of(degp):
    deg = jnp.sum(degp, axis=0)[:, 0:1] + 1.0
    return lax.rsqrt(deg)


def _tc1_body(x_ref, w_ref, degp_ref, y_ref):
    dinv = _dinv_of(degp_ref[...])
    y_ref[...] = jnp.dot(x_ref[...], w_ref[...],
                         preferred_element_type=F32) * dinv


def _tc2_body(p_ref, degp_ref, b_ref, w_ref, y_ref, *, n_valid):
    dinv = _dinv_of(degp_ref[...])
    p = p_ref[...]
    h = jnp.maximum((p[0] + p[1]) * dinv + b_ref[...], 0.0)
    y = jnp.dot(h, w_ref[...], preferred_element_type=F32) * dinv
    ridx = (pl.program_id(0) * _TC_BLK
            + lax.broadcasted_iota(I32, (_TC_BLK, 1), 0))
    y_ref[...] = jnp.where(ridx < n_valid, y, 0.0)


def _tc3_body(q_ref, degp_ref, b_ref, wd_ref, bd_ref, logp_ref, emb_ref):
    dinv = _dinv_of(degp_ref[...])
    q = q_ref[...]
    h = (q[0] + q[1]) * dinv + b_ref[...]
    nrm = jnp.sqrt(jnp.sum(h * h, axis=1, keepdims=True))
    emb = h / (nrm + 1e-12)
    logits = jnp.dot(emb, wd_ref[...], preferred_element_type=F32) + bd_ref[...]
    m = jnp.max(logits, axis=1, keepdims=True)
    lse = jnp.log(jnp.sum(jnp.exp(logits - m), axis=1, keepdims=True)) + m
    logp_ref[...] = logits - lse
    emb_ref[...] = emb


def kernel(x, edge_index, W1, b1, W2, b2, Wd, bd):
    n, in_c = x.shape
    e = edge_index.shape[1]
    h1c = W1.shape[1]
    h2c = W2.shape[1]
    oc = Wd.shape[1]

    n_pad = _ceil_to(n + _K, _N_SUB * _K)
    e_pad = _ceil_to(e, _NW * _K)
    grid = n_pad // _TC_BLK

    x_p = jnp.pad(x, ((0, n_pad - n), (0, 0)))
    pad_e = e_pad - e
    src_p = jnp.concatenate(
        [edge_index[0].astype(I32), jnp.full((pad_e,), n, I32)])
    dst_p = jnp.concatenate(
        [edge_index[1].astype(I32), jnp.full((pad_e,), n, I32)])

    degp = _make_deg(n_pad, e_pad)(dst_p)

    y1 = pl.pallas_call(
        _tc1_body,
        grid=grid,
        in_specs=[
            pl.BlockSpec((_TC_BLK, in_c), lambda i: (i, 0)),
            pl.BlockSpec((in_c, h1c), lambda i: (0, 0)),
            pl.BlockSpec((_N_CORES, _TC_BLK, _DEG_W), lambda i: (0, i, 0)),
        ],
        out_specs=pl.BlockSpec((_TC_BLK, h1c), lambda i: (i, 0)),
        out_shape=jax.ShapeDtypeStruct((n_pad, h1c), F32),
    )(x_p, W1, degp)

    p1 = _make_agg(n_pad, e_pad, h1c)(y1, src_p, dst_p)

    y2 = pl.pallas_call(
        functools.partial(_tc2_body, n_valid=n),
        grid=grid,
        in_specs=[
            pl.BlockSpec((_N_CORES, _TC_BLK, h1c), lambda i: (0, i, 0)),
            pl.BlockSpec((_N_CORES, _TC_BLK, _DEG_W), lambda i: (0, i, 0)),
            pl.BlockSpec((1, h1c), lambda i: (0, 0)),
            pl.BlockSpec((h1c, h2c), lambda i: (0, 0)),
        ],
        out_specs=pl.BlockSpec((_TC_BLK, h2c), lambda i: (i, 0)),
        out_shape=jax.ShapeDtypeStruct((n_pad, h2c), F32),
    )(p1, degp, b1.reshape(1, h1c), W2)

    p2 = _make_agg(n_pad, e_pad, h2c)(y2, src_p, dst_p)

    logp, emb = pl.pallas_call(
        _tc3_body,
        grid=grid,
        in_specs=[
            pl.BlockSpec((_N_CORES, _TC_BLK, h2c), lambda i: (0, i, 0)),
            pl.BlockSpec((_N_CORES, _TC_BLK, _DEG_W), lambda i: (0, i, 0)),
            pl.BlockSpec((1, h2c), lambda i: (0, 0)),
            pl.BlockSpec((h2c, oc), lambda i: (0, 0)),
            pl.BlockSpec((1, oc), lambda i: (0, 0)),
        ],
        out_specs=[
            pl.BlockSpec((_TC_BLK, oc), lambda i: (i, 0)),
            pl.BlockSpec((_TC_BLK, h2c), lambda i: (i, 0)),
        ],
        out_shape=[
            jax.ShapeDtypeStruct((n_pad, oc), F32),
            jax.ShapeDtypeStruct((n_pad, h2c), F32),
        ],
    )(p2, degp, b2.reshape(1, h2c), Wd, bd.reshape(1, oc))

    return logp[:n], emb[:n]
